# baseline (speedup 1.0000x reference)
import functools

import jax
import jax.numpy as jnp
from jax.experimental import pallas as pl

N = 4096
DIM_IN = 7
DIM_H = 8
BOND_CUTOFF = 3.6
CHUNK = 256


def _leaky(v):
    return jnp.where(v >= 0, v, 0.01 * v)


def _gnn_kernel(x_ref, xt3_ref, w1t_ref, b1_ref, w2t_ref, b2_ref,
                w3et_ref, b3e_ref, w3kt_ref, b3k_ref, w3qt_ref, b3q_ref,
                wext_ref, wat_ref, be_ref, wdt_ref, bd_ref, out_ref):
    x = x_ref[...]
    xt3 = xt3_ref[...]

    h = _leaky(jnp.dot(x, w1t_ref[...], preferred_element_type=jnp.float32)
               + b1_ref[...])
    h = _leaky(jnp.dot(h, w2t_ref[...], preferred_element_type=jnp.float32)
               + b2_ref[...])
    vext = jnp.dot(h, w3et_ref[...], preferred_element_type=jnp.float32) \
        + b3e_ref[...]
    kk = jnp.dot(h, w3kt_ref[...], preferred_element_type=jnp.float32) \
        + b3k_ref[...]
    qq = jnp.dot(h, w3qt_ref[...], preferred_element_type=jnp.float32) \
        + b3q_ref[...]
    s = jnp.sum(qq * kk, axis=1, keepdims=True)
    w = jnp.exp(s - jnp.max(s))
    u = w * vext

    wext = wext_ref[...]
    wat = wat_ref[...]
    be = be_ref[...]
    wdt = wdt_ref[...]
    bd = bd_ref[...]
    xt3b = xt3.astype(jnp.bfloat16)
    xb = x[:, 0:3].astype(jnp.bfloat16)
    cutoff = jnp.bfloat16(BOND_CUTOFF)

    NB = N // CHUNK
    uT = u.T
    accs = [None] * NB
    accTs = [None] * NB
    for i in range(NB):
        for j in range(i, NB):
            ci = xb[i * CHUNK:(i + 1) * CHUNK, :]
            tj = xt3b[:, j * CHUNK:(j + 1) * CHUNK]
            d = (jnp.abs(ci[:, 0:1] - tj[0:1, :])
                 + jnp.abs(ci[:, 1:2] - tj[1:2, :])
                 + jnp.abs(ci[:, 2:3] - tj[2:3, :]))
            a16 = jnp.where(d <= cutoff, jnp.bfloat16(1), jnp.bfloat16(0))
            a = a16.astype(jnp.float32)
            uj = u[j * CHUNK:(j + 1) * CHUNK, :]
            pij = jnp.dot(a, uj, preferred_element_type=jnp.float32)
            accs[i] = pij if accs[i] is None else accs[i] + pij
            if i < j:
                uiT = uT[:, i * CHUNK:(i + 1) * CHUNK]
                pjiT = jnp.dot(uiT, a, preferred_element_type=jnp.float32)
                accTs[j] = pjiT if accTs[j] is None else accTs[j] + pjiT

    for i in range(NB):
        if accTs[i] is not None:
            accs[i] = accs[i] + accTs[i].T
        r0 = i * CHUNK
        xc = x[r0:r0 + CHUNK, :]
        acc = accs[i] - u[r0:r0 + CHUNK, :]
        den = jnp.maximum(acc[:, 7:8], 1e-30)
        pre = (jnp.dot(xc, wext, preferred_element_type=jnp.float32)
               + jnp.dot(acc, wat, preferred_element_type=jnp.float32) / den
               + be)
        codes = _leaky(pre)
        out_ref[r0:r0 + CHUNK, :] = (
            jnp.dot(codes, wdt, preferred_element_type=jnp.float32) + bd)


@functools.partial(jax.jit, static_argnames=("interpret",))
def kernel(x, W1, b1, W2, b2, W3, b3, We, be, Wd, bd, interpret=False):
    xt3 = x[:, :3].T
    w3v = W3[:DIM_IN]
    w3et = jnp.concatenate([w3v, jnp.zeros((1, DIM_H), jnp.float32)], 0).T
    b3e = jnp.concatenate([b3[:DIM_IN], jnp.ones((1,), jnp.float32)])
    w3kt = W3[DIM_IN:DIM_IN + DIM_H].T
    b3k = b3[DIM_IN:DIM_IN + DIM_H]
    w3qt = W3[DIM_IN + DIM_H:].T
    b3q = b3[DIM_IN + DIM_H:]
    wext = We[:, :DIM_IN].T
    wa = We[:, DIM_IN:]
    wat = jnp.concatenate([wa, jnp.zeros((DIM_H, 1), jnp.float32)], 1).T
    wdt = Wd.T

    r2 = lambda v: v.reshape(1, -1)
    return pl.pallas_call(
        _gnn_kernel,
        out_shape=jax.ShapeDtypeStruct((N, DIM_IN), jnp.float32),
        interpret=interpret,
    )(x, xt3, W1.T, r2(b1), W2.T, r2(b2),
      w3et, r2(b3e), w3kt, r2(b3k), w3qt, r2(b3q),
      wext, wat, r2(be), wdt, r2(bd))


# speedup vs baseline: 1.0278x; 1.0278x over previous
import functools

import jax
import jax.numpy as jnp
from jax.experimental import pallas as pl

N = 4096
DIM_IN = 7
DIM_H = 8
BOND_CUTOFF = 3.6
CHUNK = 512


def _leaky(v):
    return jnp.where(v >= 0, v, 0.01 * v)


def _gnn_kernel(x_ref, xt3_ref, w1t_ref, b1_ref, w2t_ref, b2_ref,
                w3et_ref, b3e_ref, w3kt_ref, b3k_ref, w3qt_ref, b3q_ref,
                wext_ref, wat_ref, be_ref, wdt_ref, bd_ref, out_ref):
    x = x_ref[...]
    xt3 = xt3_ref[...]

    h = _leaky(jnp.dot(x, w1t_ref[...], preferred_element_type=jnp.float32)
               + b1_ref[...])
    h = _leaky(jnp.dot(h, w2t_ref[...], preferred_element_type=jnp.float32)
               + b2_ref[...])
    vext = jnp.dot(h, w3et_ref[...], preferred_element_type=jnp.float32) \
        + b3e_ref[...]
    kk = jnp.dot(h, w3kt_ref[...], preferred_element_type=jnp.float32) \
        + b3k_ref[...]
    qq = jnp.dot(h, w3qt_ref[...], preferred_element_type=jnp.float32) \
        + b3q_ref[...]
    s = jnp.sum(qq * kk, axis=1, keepdims=True)
    w = jnp.exp(s - jnp.max(s))
    u = w * vext

    wext = wext_ref[...]
    wat = wat_ref[...]
    be = be_ref[...]
    wdt = wdt_ref[...]
    bd = bd_ref[...]
    xt3b = xt3.astype(jnp.bfloat16)
    xb = x[:, 0:3].astype(jnp.bfloat16)
    cutoff = jnp.bfloat16(BOND_CUTOFF)

    NB = N // CHUNK
    uT = u.T
    accs = [None] * NB
    accTs = [None] * NB
    for i in range(NB):
        for j in range(i, NB):
            ci = xb[i * CHUNK:(i + 1) * CHUNK, :]
            tj = xt3b[:, j * CHUNK:(j + 1) * CHUNK]
            d = (jnp.abs(ci[:, 0:1] - tj[0:1, :])
                 + jnp.abs(ci[:, 1:2] - tj[1:2, :])
                 + jnp.abs(ci[:, 2:3] - tj[2:3, :]))
            a16 = jnp.where(d <= cutoff, jnp.bfloat16(1), jnp.bfloat16(0))
            a = a16.astype(jnp.float32)
            uj = u[j * CHUNK:(j + 1) * CHUNK, :]
            pij = jnp.dot(a, uj, preferred_element_type=jnp.float32)
            accs[i] = pij if accs[i] is None else accs[i] + pij
            if i < j:
                uiT = uT[:, i * CHUNK:(i + 1) * CHUNK]
                pjiT = jnp.dot(uiT, a, preferred_element_type=jnp.float32)
                accTs[j] = pjiT if accTs[j] is None else accTs[j] + pjiT

    for i in range(NB):
        if accTs[i] is not None:
            accs[i] = accs[i] + accTs[i].T
        r0 = i * CHUNK
        xc = x[r0:r0 + CHUNK, :]
        acc = accs[i] - u[r0:r0 + CHUNK, :]
        den = jnp.maximum(acc[:, 7:8], 1e-30)
        pre = (jnp.dot(xc, wext, preferred_element_type=jnp.float32)
               + jnp.dot(acc, wat, preferred_element_type=jnp.float32) / den
               + be)
        codes = _leaky(pre)
        out_ref[r0:r0 + CHUNK, :] = (
            jnp.dot(codes, wdt, preferred_element_type=jnp.float32) + bd)


@functools.partial(jax.jit, static_argnames=("interpret",))
def kernel(x, W1, b1, W2, b2, W3, b3, We, be, Wd, bd, interpret=False):
    xt3 = x[:, :3].T
    w3v = W3[:DIM_IN]
    w3et = jnp.concatenate([w3v, jnp.zeros((1, DIM_H), jnp.float32)], 0).T
    b3e = jnp.concatenate([b3[:DIM_IN], jnp.ones((1,), jnp.float32)])
    w3kt = W3[DIM_IN:DIM_IN + DIM_H].T
    b3k = b3[DIM_IN:DIM_IN + DIM_H]
    w3qt = W3[DIM_IN + DIM_H:].T
    b3q = b3[DIM_IN + DIM_H:]
    wext = We[:, :DIM_IN].T
    wa = We[:, DIM_IN:]
    wat = jnp.concatenate([wa, jnp.zeros((DIM_H, 1), jnp.float32)], 1).T
    wdt = Wd.T

    r2 = lambda v: v.reshape(1, -1)
    return pl.pallas_call(
        _gnn_kernel,
        out_shape=jax.ShapeDtypeStruct((N, DIM_IN), jnp.float32),
        interpret=interpret,
    )(x, xt3, W1.T, r2(b1), W2.T, r2(b2),
      w3et, r2(b3e), w3kt, r2(b3k), w3qt, r2(b3q),
      wext, wat, r2(be), wdt, r2(bd))
